# baseline (speedup 1.0000x reference)
import functools

import jax
import jax.numpy as jnp
from jax import lax
from jax.experimental import pallas as pl
from jax.experimental.pallas import tpu as pltpu
from jax.experimental.pallas import tpu_sc as plsc

B = 4096
D = 64
NSC = 2048
NTC = B - NSC
UNROLL = 8

NC = 2
NS = 16
NW = NC * NS
BPW = NSC // NW

_mesh = plsc.VectorSubcoreMesh(core_axis_name="c", subcore_axis_name="s")


@functools.partial(
    pl.kernel,
    mesh=_mesh,
    out_type=jax.ShapeDtypeStruct((NSC, D), jnp.float32),
    scratch_types=[
        pltpu.VMEM((BPW,), jnp.int32),
        pltpu.VMEM((BPW, D), jnp.float32),
        pltpu.SemaphoreType.DMA,
        pltpu.SemaphoreType.DMA,
    ],
)
def _sc_gather(nodes_hbm, table_hbm, out_hbm, idx_v, rows_v, isem, sem):
    wid = lax.axis_index("s") * NC + lax.axis_index("c")
    base = wid * BPW
    pltpu.async_copy(nodes_hbm.at[pl.ds(base, BPW)], idx_v, isem).wait()

    def chunk(cb, _):
        vals = idx_v[pl.ds(cb * 16, 16)]
        for t in range(16):
            pltpu.async_copy(
                table_hbm.at[pl.ds(vals[t], 1), :],
                rows_v.at[pl.ds(cb * 16 + t, 1), :], sem)
        return 0

    lax.fori_loop(0, BPW // 16, chunk, 0)
    pltpu.make_async_copy(table_hbm.at[pl.ds(0, BPW), :], rows_v, sem).wait()
    pltpu.sync_copy(rows_v, out_hbm.at[pl.ds(base, BPW)])


def _tc_body(idx_s, table_hbm, out_ref, rows_v, sem):
    def issue(jb, _):
        for u in range(UNROLL):
            j = jb * UNROLL + u
            pltpu.make_async_copy(
                table_hbm.at[pl.ds(idx_s[j], 1), :],
                rows_v.at[pl.ds(j, 1), :], sem).start()
        return 0

    lax.fori_loop(0, NTC // UNROLL, issue, 0)

    def drain(jb, _):
        for u in range(UNROLL):
            j = jb * UNROLL + u
            pltpu.make_async_copy(
                table_hbm.at[pl.ds(0, 1), :],
                rows_v.at[pl.ds(j, 1), :], sem).wait()
        return 0

    lax.fori_loop(0, NTC // UNROLL, drain, 0)

    x = rows_v[...]
    rinv = lax.rsqrt(jnp.sum(x * x, axis=1, keepdims=True))
    out_ref[...] = (x * rinv).T


def _tc_gather_norm(nodes_tc, table):
    grid_spec = pltpu.PrefetchScalarGridSpec(
        num_scalar_prefetch=1,
        grid=(1,),
        in_specs=[pl.BlockSpec(memory_space=pl.ANY)],
        out_specs=pl.BlockSpec((D, NTC), lambda i, idx: (0, 0)),
        scratch_shapes=[
            pltpu.VMEM((NTC, D), jnp.float32),
            pltpu.SemaphoreType.DMA,
        ],
    )
    return pl.pallas_call(
        _tc_body,
        grid_spec=grid_spec,
        out_shape=jax.ShapeDtypeStruct((D, NTC), jnp.float32),
    )(nodes_tc, table)


def _norm_t_body(rows_ref, out_ref):
    x = rows_ref[...]
    rinv = lax.rsqrt(jnp.sum(x * x, axis=1, keepdims=True))
    out_ref[...] = (x * rinv).T


def _norm_t(rows):
    return pl.pallas_call(
        _norm_t_body,
        out_shape=jax.ShapeDtypeStruct((D, NSC), jnp.float32),
    )(rows)


def kernel(nodes, table):
    nodes = nodes.astype(jnp.int32)
    rows_sc = _sc_gather(nodes[:NSC], table)
    out_tc = _tc_gather_norm(nodes[NSC:], table)
    out_sc = _norm_t(rows_sc)
    return jnp.concatenate([out_sc, out_tc], axis=1)


# speedup vs baseline: 1.1957x; 1.1957x over previous
import jax
import jax.numpy as jnp
from jax import lax
from jax.experimental import pallas as pl
from jax.experimental.pallas import tpu as pltpu

B = 4096
D = 64
UNROLL = 8


def _body(idx_s, table_hbm, out_ref, rows_v, sem):
    def issue(jb, _):
        for u in range(UNROLL):
            j = jb * UNROLL + u
            pltpu.make_async_copy(
                table_hbm.at[pl.ds(idx_s[j], 1), :],
                rows_v.at[pl.ds(j, 1), :], sem).start()
        return 0

    lax.fori_loop(0, B // UNROLL, issue, 0)
    pltpu.make_async_copy(table_hbm.at[pl.ds(0, B), :], rows_v, sem).wait()

    x = rows_v[...]
    rinv = lax.rsqrt(jnp.sum(x * x, axis=1, keepdims=True))
    out_ref[...] = (x * rinv).T


def kernel(nodes, table):
    grid_spec = pltpu.PrefetchScalarGridSpec(
        num_scalar_prefetch=1,
        grid=(1,),
        in_specs=[pl.BlockSpec(memory_space=pl.ANY)],
        out_specs=pl.BlockSpec((D, B), lambda i, idx: (0, 0)),
        scratch_shapes=[
            pltpu.VMEM((B, D), jnp.float32),
            pltpu.SemaphoreType.DMA,
        ],
    )
    return pl.pallas_call(
        _body,
        grid_spec=grid_spec,
        out_shape=jax.ShapeDtypeStruct((D, B), jnp.float32),
    )(nodes.astype(jnp.int32), table)
